# Initial kernel scaffold; baseline (speedup 1.0000x reference)
import jax
import jax.numpy as jnp
from jax.experimental import pallas as pl


def kernel(x_t, cond, t, edge_index, edge_dist, Wn1, bn1, Wn2, bn2, Wc1, bc1, Wc2, We1, be1, We2, be2):
    raise NotImplementedError("write your pallas kernel here")


# speedup vs baseline: 5.2789x; 5.2789x over previous
import functools

import jax
import jax.numpy as jnp
from jax import lax
from jax.experimental import pallas as pl
from jax.experimental.pallas import tpu as pltpu
from jax.experimental.pallas import tpu_sc as plsc

N = 10000
E = 320000
H = 128
NC, NS, LANES = 2, 16, 16
NW = NC * NS
EPW = E // NW
GC = 80
NGC = EPW // GC
SC2 = 2000
NSC2 = EPW // SC2
G16 = SC2 // LANES

_SC_MESH = plsc.VectorSubcoreMesh(core_axis_name="c", subcore_axis_name="s")


def _precompute_body(h_ref, wc1_ref, bc1_ref, be2_ref, ps_ref, pd_ref):
    h = h_ref[...]
    wa = wc1_ref[0:128, :]
    wb = wc1_ref[128:256, :]
    wc = wc1_ref[256:288, :]
    ps_ref[...] = jnp.dot(h, wa, preferred_element_type=jnp.float32)
    ball = bc1_ref[...] + jnp.dot(be2_ref[...], wc,
                                  preferred_element_type=jnp.float32)
    pd_ref[...] = jnp.dot(h, wb, preferred_element_type=jnp.float32) + ball


_precompute = pl.pallas_call(
    _precompute_body,
    out_shape=[jax.ShapeDtypeStruct((N, H), jnp.float32),
               jax.ShapeDtypeStruct((N, H), jnp.float32)],
)


@functools.partial(
    pl.kernel,
    out_type=[jax.ShapeDtypeStruct((E, H), jnp.float32),
              jax.ShapeDtypeStruct((E, H), jnp.float32)],
    mesh=_SC_MESH,
    scratch_types=[
        pltpu.VMEM((GC,), jnp.int32),
        pltpu.VMEM((GC,), jnp.int32),
        pltpu.VMEM((GC, H), jnp.float32),
        pltpu.VMEM((GC, H), jnp.float32),
        pltpu.SemaphoreType.DMA,
        pltpu.SemaphoreType.DMA,
    ],
)
def _gather_kernel(ps_hbm, pd_hbm, src_hbm, dst_hbm, gs_hbm, gd_hbm,
                   idxs_v, idxd_v, bufs_v, bufd_v, sem_s, sem_d):
    wid = lax.axis_index("s") * NC + lax.axis_index("c")
    base = wid * EPW

    def body(i, carry):
        off = base + i * GC
        pltpu.sync_copy(src_hbm.at[pl.ds(off, GC)], idxs_v)
        pltpu.sync_copy(dst_hbm.at[pl.ds(off, GC)], idxd_v)
        cp_s = pltpu.async_copy(ps_hbm.at[idxs_v], bufs_v, sem_s)
        cp_d = pltpu.async_copy(pd_hbm.at[idxd_v], bufd_v, sem_d)
        cp_s.wait()
        cp_d.wait()
        pltpu.sync_copy(bufs_v, gs_hbm.at[pl.ds(off, GC)])
        pltpu.sync_copy(bufd_v, gd_hbm.at[pl.ds(off, GC)])
        return carry

    lax.fori_loop(0, NGC, body, 0)


BE = 4000


def _edge_body(gs_ref, gd_ref, d_ref, we1_ref, be1_ref, we2_ref, wc1c_ref,
               wc2_ref, c_ref):
    d = d_ref[...]
    a = jnp.dot(d, we1_ref[...],
                preferred_element_type=jnp.float32) + be1_ref[...]
    sa = a * jax.nn.sigmoid(a)
    wcomb = jnp.dot(we2_ref[...], wc1c_ref[...],
                    preferred_element_type=jnp.float32)
    q = jnp.dot(sa, wcomb, preferred_element_type=jnp.float32)
    u = gs_ref[...] + gd_ref[...] + q
    su = u * jax.nn.sigmoid(u)
    c_ref[...] = jnp.dot(su, wc2_ref[...], preferred_element_type=jnp.float32)


_edge_epilogue = pl.pallas_call(
    _edge_body,
    grid=(E // BE,),
    in_specs=[
        pl.BlockSpec((BE, H), lambda i: (i, 0)),
        pl.BlockSpec((BE, H), lambda i: (i, 0)),
        pl.BlockSpec((BE, 1), lambda i: (i, 0)),
        pl.BlockSpec((1, 32), lambda i: (0, 0)),
        pl.BlockSpec((1, 32), lambda i: (0, 0)),
        pl.BlockSpec((32, 32), lambda i: (0, 0)),
        pl.BlockSpec((32, H), lambda i: (0, 0)),
        pl.BlockSpec((H, 1), lambda i: (0, 0)),
    ],
    out_specs=pl.BlockSpec((BE, 1), lambda i: (i, 0)),
    out_shape=jax.ShapeDtypeStruct((E, 1), jnp.float32),
)


@functools.partial(
    pl.kernel,
    out_type=jax.ShapeDtypeStruct((NW * 3 * N,), jnp.float32),
    mesh=_SC_MESH,
    scratch_types=[
        pltpu.VMEM((N,), jnp.float32),
        pltpu.VMEM((N,), jnp.float32),
        pltpu.VMEM((N,), jnp.float32),
        pltpu.VMEM((N,), jnp.float32),
        pltpu.VMEM((N,), jnp.float32),
        pltpu.VMEM((N,), jnp.float32),
        pltpu.VMEM((SC2,), jnp.int32),
        pltpu.VMEM((SC2,), jnp.int32),
        pltpu.VMEM((SC2,), jnp.float32),
    ],
    compiler_params=pltpu.CompilerParams(needs_layout_passes=False),
)
def _scatter_kernel(xt_hbm, src_hbm, dst_hbm, c_hbm, out_hbm,
                    xv, yv, zv, ax, ay, az, sv, dv, cv):
    wid = lax.axis_index("s") * NC + lax.axis_index("c")
    base = wid * EPW
    pltpu.sync_copy(xt_hbm.at[pl.ds(0, N)], xv)
    pltpu.sync_copy(xt_hbm.at[pl.ds(N, N)], yv)
    pltpu.sync_copy(xt_hbm.at[pl.ds(2 * N, N)], zv)
    zeros = jnp.zeros((LANES,), jnp.float32)

    def zbody(i, carry):
        ax[pl.ds(i * LANES, LANES)] = zeros
        ay[pl.ds(i * LANES, LANES)] = zeros
        az[pl.ds(i * LANES, LANES)] = zeros
        return carry

    lax.fori_loop(0, N // LANES, zbody, 0)

    def chunk(ci, carry):
        off = base + ci * SC2
        pltpu.sync_copy(src_hbm.at[pl.ds(off, SC2)], sv)
        pltpu.sync_copy(dst_hbm.at[pl.ds(off, SC2)], dv)
        pltpu.sync_copy(c_hbm.at[pl.ds(off, SC2)], cv)

        def grp(g, c2):
            s = sv[pl.ds(g * LANES, LANES)]
            dd = dv[pl.ds(g * LANES, LANES)]
            xs = plsc.load_gather(xv, [s])
            xd = plsc.load_gather(xv, [dd])
            ys = plsc.load_gather(yv, [s])
            yd = plsc.load_gather(yv, [dd])
            zs = plsc.load_gather(zv, [s])
            zd = plsc.load_gather(zv, [dd])
            dx = xs - xd
            dy = ys - yd
            dz = zs - zd
            n2 = jnp.maximum(dx * dx + dy * dy + dz * dz,
                             jnp.float32(1e-16))
            ib = plsc.bitcast(n2, jnp.int32)
            yb = jnp.int32(0x5F3759DF) - lax.shift_right_logical(ib, 1)
            yr = plsc.bitcast(yb, jnp.float32)
            yr = yr * (1.5 - 0.5 * n2 * yr * yr)
            yr = yr * (1.5 - 0.5 * n2 * yr * yr)
            yr = yr * (1.5 - 0.5 * n2 * yr * yr)
            cc = cv[pl.ds(g * LANES, LANES)]
            s_c = cc * yr
            plsc.addupdate_scatter(ax, [dd], s_c * dx)
            plsc.addupdate_scatter(ay, [dd], s_c * dy)
            plsc.addupdate_scatter(az, [dd], s_c * dz)
            return c2

        lax.fori_loop(0, G16, grp, 0)
        return carry

    lax.fori_loop(0, NSC2, chunk, 0)
    obase = wid * (3 * N)
    pltpu.sync_copy(ax, out_hbm.at[pl.ds(obase, N)])
    pltpu.sync_copy(ay, out_hbm.at[pl.ds(obase + N, N)])
    pltpu.sync_copy(az, out_hbm.at[pl.ds(obase + 2 * N, N)])


def _reduce_body(part_ref, xt_ref, o_ref):
    o_ref[...] = xt_ref[...] + jnp.sum(part_ref[...], axis=0)


_reduce = pl.pallas_call(
    _reduce_body,
    out_shape=jax.ShapeDtypeStruct((3, N), jnp.float32),
)


def kernel(x_t, cond, t, edge_index, edge_dist, Wn1, bn1, Wn2, bn2,
           Wc1, bc1, Wc2, We1, be1, We2, be2):
    B = x_t.shape[0]
    h = jnp.concatenate(
        [cond.reshape(N, H - 1),
         jnp.full((N, 1), t, dtype=jnp.float32)], axis=1)
    src = edge_index[0]
    dst = edge_index[1]
    x3n = x_t.reshape(N, 3).T

    ps, pd = _precompute(h, Wc1, bc1.reshape(1, H), be2.reshape(1, 32))
    gs, gd = _gather_kernel(ps, pd, src, dst)
    c = _edge_epilogue(gs, gd, edge_dist.reshape(E, 1), We1.reshape(1, 32),
                       be1.reshape(1, 32), We2, Wc1[256:288], Wc2)
    partials = _scatter_kernel(x3n.reshape(3 * N), src, dst, c.reshape(E))
    out3n = _reduce(partials.reshape(NW, 3, N), x3n)
    return out3n.T.reshape(B, N, 3)
